# baseline (speedup 1.0000x reference)
import functools

import jax
import jax.numpy as jnp
from jax.experimental import pallas as pl
from jax.experimental.pallas import tpu as pltpu

_TOP_K = 4


def _gmm_kernel(off_r, end_r, tid_r, eid_r, x_ref, fcg_ref, fcx_ref,
                proj_ref, out_ref, *, tm, sb):
    g = pl.program_id(0)
    h = pl.program_id(1)
    prev_tid = tid_r[jnp.maximum(g - 1, 0)]
    first = (h == 0) & ((g == 0) | (tid_r[g] != prev_tid))

    @pl.when(first)
    def _():
        out_ref[...] = jnp.zeros_like(out_ref)

    base = tid_r[g] * tm
    lo = off_r[g]
    hi = end_r[g]
    fcg = fcg_ref[0].astype(jnp.bfloat16)
    fcx = fcx_ref[0].astype(jnp.bfloat16)
    proj = proj_ref[0].astype(jnp.bfloat16)
    for b in range(tm // sb):
        sb_lo = base + b * sb

        @pl.when((hi > sb_lo) & (lo < sb_lo + sb))
        def _():
            xb = x_ref[pl.ds(b * sb, sb), :].astype(jnp.bfloat16)
            gg = jnp.dot(xb, fcg, preferred_element_type=jnp.float32)
            hh = jnp.dot(xb, fcx, preferred_element_type=jnp.float32)
            act = (gg * jax.nn.sigmoid(gg)) * hh
            row = sb_lo + jax.lax.broadcasted_iota(jnp.int32, (sb, 1), 0)
            mask = (row >= lo) & (row < hi)
            act = jnp.where(mask, act, 0.0).astype(jnp.bfloat16)
            contrib = jnp.dot(act, proj, preferred_element_type=jnp.float32)
            out_ref[pl.ds(b * sb, sb), :] += contrib


def _dense_ffn_kernel(x_ref, fcg_ref, fcx_ref, proj_ref, out_ref):
    h = pl.program_id(1)
    xb = x_ref[...].astype(jnp.bfloat16)
    gg = jnp.dot(xb, fcg_ref[...].astype(jnp.bfloat16),
                 preferred_element_type=jnp.float32)
    hh = jnp.dot(xb, fcx_ref[...].astype(jnp.bfloat16),
                 preferred_element_type=jnp.float32)
    act = ((gg * jax.nn.sigmoid(gg)) * hh).astype(jnp.bfloat16)
    contrib = jnp.dot(act, proj_ref[...].astype(jnp.bfloat16),
                      preferred_element_type=jnp.float32)

    @pl.when(h == 0)
    def _():
        out_ref[...] = contrib

    @pl.when(h != 0)
    def _():
        out_ref[...] += contrib


def kernel(x, shared_fc, shared_proj, experts_fc, experts_proj, gate_w,
           expert_bias):
    Bq, Tq, C = x.shape
    E = experts_fc.shape[0]
    HID = experts_proj.shape[1]
    K = _TOP_K
    N = Bq * Tq
    S = N * K
    i32 = jnp.int32

    TM = min(2048, S)
    SB = min(128, TM)
    HB = min(512, HID)
    assert S % TM == 0 and HID % HB == 0
    NT = S // TM
    NH = HID // HB
    G = NT + E - 1

    flat_x = x.reshape(N, C)

    logits = flat_x @ gate_w + expert_bias
    gw = jax.nn.sigmoid(logits)
    top_w, top_i = jax.lax.top_k(gw, K)
    top_w = top_w / jnp.sum(top_w, axis=-1, keepdims=True)
    top_i = top_i.astype(i32)

    onehot = jax.nn.one_hot(top_i, E, dtype=jnp.float32).sum(axis=1)
    csum = jnp.cumsum(onehot, axis=0)
    rank = jnp.take_along_axis((csum - onehot), top_i, axis=1)
    sizes = csum[-1].astype(i32)
    offsets = jnp.concatenate(
        [jnp.zeros((1,), i32), jnp.cumsum(sizes).astype(i32)])
    pos = jnp.take(offsets, top_i) + rank.astype(i32)

    pos_flat = pos.reshape(-1)
    tok_sorted = jnp.zeros((S,), i32).at[pos_flat].set(
        jnp.repeat(jnp.arange(N, dtype=i32), K))
    x_sorted = jnp.take(flat_x, tok_sorted, axis=0)

    first_tile = offsets[:E] // TM
    last_tile = (offsets[1:] - 1) // TM
    n_t = jnp.where(sizes > 0, last_tile - first_tile + 1, 0).astype(i32)
    cum_nt = jnp.cumsum(n_t)
    items_before = cum_nt - n_t
    total = cum_nt[-1]

    i = jnp.arange(G, dtype=i32)
    e_of = jnp.searchsorted(cum_nt, i, side='right').astype(i32)
    valid = i < total
    e_idx = jnp.minimum(e_of, E - 1)
    tile_ids = jnp.where(valid, first_tile[e_idx] + (i - items_before[e_idx]),
                         NT - 1).astype(i32)
    expert_ids = jnp.where(valid, e_idx, 0).astype(i32)
    off_arr = jnp.where(valid, offsets[e_idx], S).astype(i32)
    end_arr = jnp.where(valid, offsets[e_idx + 1], S).astype(i32)

    out_sorted = pl.pallas_call(
        functools.partial(_gmm_kernel, tm=TM, sb=SB),
        grid_spec=pltpu.PrefetchScalarGridSpec(
            num_scalar_prefetch=4,
            grid=(G, NH),
            in_specs=[
                pl.BlockSpec((TM, C),
                             lambda g, h, off, end, tid, eid: (tid[g], 0)),
                pl.BlockSpec((1, C, HB),
                             lambda g, h, off, end, tid, eid: (eid[g], 0, h)),
                pl.BlockSpec((1, C, HB),
                             lambda g, h, off, end, tid, eid:
                             (eid[g], 0, h + NH)),
                pl.BlockSpec((1, HB, C),
                             lambda g, h, off, end, tid, eid: (eid[g], h, 0)),
            ],
            out_specs=pl.BlockSpec((TM, C),
                                   lambda g, h, off, end, tid, eid:
                                   (tid[g], 0)),
        ),
        out_shape=jax.ShapeDtypeStruct((S, C), jnp.float32),
    )(off_arr, end_arr, tile_ids, expert_ids, x_sorted,
      experts_fc, experts_fc, experts_proj)

    TMS = min(512, N)
    NTS = N // TMS
    shared_out = pl.pallas_call(
        _dense_ffn_kernel,
        grid=(NTS, NH),
        in_specs=[
            pl.BlockSpec((TMS, C), lambda t, h: (t, 0)),
            pl.BlockSpec((C, HB), lambda t, h: (0, h)),
            pl.BlockSpec((C, HB), lambda t, h: (0, h + NH)),
            pl.BlockSpec((HB, C), lambda t, h: (h, 0)),
        ],
        out_specs=pl.BlockSpec((TMS, C), lambda t, h: (t, 0)),
        out_shape=jax.ShapeDtypeStruct((N, C), jnp.float32),
    )(flat_x, shared_fc, shared_fc, shared_proj)

    routed = jnp.sum(out_sorted[pos] * top_w[..., None], axis=1)

    return (shared_out + routed).reshape(Bq, Tq, C)


# speedup vs baseline: 1.1111x; 1.1111x over previous
import functools

import jax
import jax.numpy as jnp
from jax.experimental import pallas as pl
from jax.experimental.pallas import tpu as pltpu

_TOP_K = 4


def _gmm_kernel(off_r, end_r, tid_r, eid_r, x_ref, fcg_ref, fcx_ref,
                proj_ref, out_ref, *, tm):
    g = pl.program_id(0)
    h = pl.program_id(1)
    xb = x_ref[...].astype(jnp.bfloat16)
    gg = jnp.dot(xb, fcg_ref[0].astype(jnp.bfloat16),
                 preferred_element_type=jnp.float32)
    hh = jnp.dot(xb, fcx_ref[0].astype(jnp.bfloat16),
                 preferred_element_type=jnp.float32)
    act = (gg * jax.nn.sigmoid(gg)) * hh
    row = tid_r[g] * tm + jax.lax.broadcasted_iota(jnp.int32, (tm, 1), 0)
    mask = (row >= off_r[g]) & (row < end_r[g])
    act = jnp.where(mask, act, 0.0).astype(jnp.bfloat16)
    contrib = jnp.dot(act, proj_ref[0].astype(jnp.bfloat16),
                      preferred_element_type=jnp.float32)
    prev_tid = tid_r[jnp.maximum(g - 1, 0)]
    first = (h == 0) & ((g == 0) | (tid_r[g] != prev_tid))

    @pl.when(first)
    def _():
        out_ref[...] = contrib

    @pl.when(jnp.logical_not(first))
    def _():
        out_ref[...] += contrib


def _dense_ffn_kernel(x_ref, fcg_ref, fcx_ref, proj_ref, out_ref):
    h = pl.program_id(1)
    xb = x_ref[...].astype(jnp.bfloat16)
    gg = jnp.dot(xb, fcg_ref[...].astype(jnp.bfloat16),
                 preferred_element_type=jnp.float32)
    hh = jnp.dot(xb, fcx_ref[...].astype(jnp.bfloat16),
                 preferred_element_type=jnp.float32)
    act = ((gg * jax.nn.sigmoid(gg)) * hh).astype(jnp.bfloat16)
    contrib = jnp.dot(act, proj_ref[...].astype(jnp.bfloat16),
                      preferred_element_type=jnp.float32)

    @pl.when(h == 0)
    def _():
        out_ref[...] = contrib

    @pl.when(h != 0)
    def _():
        out_ref[...] += contrib


def kernel(x, shared_fc, shared_proj, experts_fc, experts_proj, gate_w,
           expert_bias):
    Bq, Tq, C = x.shape
    E = experts_fc.shape[0]
    HID = experts_proj.shape[1]
    K = _TOP_K
    N = Bq * Tq
    S = N * K
    i32 = jnp.int32

    TM = min(512, S)
    HB = min(512, HID)
    assert S % TM == 0 and HID % HB == 0
    NT = S // TM
    NH = HID // HB
    G = NT + E - 1

    flat_x = x.reshape(N, C)

    logits = flat_x @ gate_w + expert_bias
    gw = jax.nn.sigmoid(logits)
    top_w, top_i = jax.lax.top_k(gw, K)
    top_w = top_w / jnp.sum(top_w, axis=-1, keepdims=True)
    top_i = top_i.astype(i32)

    onehot = jax.nn.one_hot(top_i, E, dtype=jnp.float32).sum(axis=1)
    csum = jnp.cumsum(onehot, axis=0)
    rank = jnp.take_along_axis((csum - onehot), top_i, axis=1)
    sizes = csum[-1].astype(i32)
    offsets = jnp.concatenate(
        [jnp.zeros((1,), i32), jnp.cumsum(sizes).astype(i32)])
    pos = jnp.take(offsets, top_i) + rank.astype(i32)

    pos_flat = pos.reshape(-1)
    tok_sorted = jnp.zeros((S,), i32).at[pos_flat].set(
        jnp.repeat(jnp.arange(N, dtype=i32), K))
    x_sorted = jnp.take(flat_x, tok_sorted, axis=0)

    first_tile = offsets[:E] // TM
    last_tile = (offsets[1:] - 1) // TM
    n_t = jnp.where(sizes > 0, last_tile - first_tile + 1, 0).astype(i32)
    cum_nt = jnp.cumsum(n_t)
    items_before = cum_nt - n_t
    total = cum_nt[-1]

    i = jnp.arange(G, dtype=i32)
    e_of = jnp.searchsorted(cum_nt, i, side='right').astype(i32)
    valid = i < total
    e_idx = jnp.minimum(e_of, E - 1)
    tile_ids = jnp.where(valid, first_tile[e_idx] + (i - items_before[e_idx]),
                         NT - 1).astype(i32)
    expert_ids = jnp.where(valid, e_idx, 0).astype(i32)
    off_arr = jnp.where(valid, offsets[e_idx], S).astype(i32)
    end_arr = jnp.where(valid, offsets[e_idx + 1], S).astype(i32)

    out_sorted = pl.pallas_call(
        functools.partial(_gmm_kernel, tm=TM),
        grid_spec=pltpu.PrefetchScalarGridSpec(
            num_scalar_prefetch=4,
            grid=(G, NH),
            in_specs=[
                pl.BlockSpec((TM, C),
                             lambda g, h, off, end, tid, eid: (tid[g], 0)),
                pl.BlockSpec((1, C, HB),
                             lambda g, h, off, end, tid, eid: (eid[g], 0, h)),
                pl.BlockSpec((1, C, HB),
                             lambda g, h, off, end, tid, eid:
                             (eid[g], 0, h + NH)),
                pl.BlockSpec((1, HB, C),
                             lambda g, h, off, end, tid, eid: (eid[g], h, 0)),
            ],
            out_specs=pl.BlockSpec((TM, C),
                                   lambda g, h, off, end, tid, eid:
                                   (tid[g], 0)),
        ),
        out_shape=jax.ShapeDtypeStruct((S, C), jnp.float32),
    )(off_arr, end_arr, tile_ids, expert_ids, x_sorted,
      experts_fc, experts_fc, experts_proj)

    TMS = min(1024, N)
    NTS = N // TMS
    HBS = min(1024, HID)
    NHS = HID // HBS
    shared_out = pl.pallas_call(
        _dense_ffn_kernel,
        grid=(NTS, NHS),
        in_specs=[
            pl.BlockSpec((TMS, C), lambda t, h: (t, 0)),
            pl.BlockSpec((C, HBS), lambda t, h: (0, h)),
            pl.BlockSpec((C, HBS), lambda t, h: (0, h + NHS)),
            pl.BlockSpec((HBS, C), lambda t, h: (h, 0)),
        ],
        out_specs=pl.BlockSpec((TMS, C), lambda t, h: (t, 0)),
        out_shape=jax.ShapeDtypeStruct((N, C), jnp.float32),
    )(flat_x, shared_fc, shared_fc, shared_proj)

    routed = jnp.sum(out_sorted[pos] * top_w[..., None], axis=1)

    return (shared_out + routed).reshape(Bq, Tq, C)


# speedup vs baseline: 1.2156x; 1.0940x over previous
import functools

import jax
import jax.numpy as jnp
from jax.experimental import pallas as pl
from jax.experimental.pallas import tpu as pltpu

_TOP_K = 4


def _gmm_kernel(off_r, end_r, tid_r, eid_r, x_ref, fcg_ref, fcx_ref,
                proj_ref, out_ref, *, tm):
    g = pl.program_id(0)
    h = pl.program_id(1)
    xb = x_ref[...].astype(jnp.bfloat16)
    gg = jnp.dot(xb, fcg_ref[0].astype(jnp.bfloat16),
                 preferred_element_type=jnp.float32)
    hh = jnp.dot(xb, fcx_ref[0].astype(jnp.bfloat16),
                 preferred_element_type=jnp.float32)
    act = (gg * jax.nn.sigmoid(gg)) * hh
    row = tid_r[g] * tm + jax.lax.broadcasted_iota(jnp.int32, (tm, 1), 0)
    mask = (row >= off_r[g]) & (row < end_r[g])
    act = jnp.where(mask, act, 0.0).astype(jnp.bfloat16)
    contrib = jnp.dot(act, proj_ref[0].astype(jnp.bfloat16),
                      preferred_element_type=jnp.float32)
    prev_tid = tid_r[jnp.maximum(g - 1, 0)]
    first = (h == 0) & ((g == 0) | (tid_r[g] != prev_tid))

    @pl.when(first)
    def _():
        out_ref[...] = contrib

    @pl.when(jnp.logical_not(first))
    def _():
        out_ref[...] += contrib


def _dense_ffn_kernel(x_ref, fcg_ref, fcx_ref, proj_ref, out_ref):
    h = pl.program_id(1)
    xb = x_ref[...].astype(jnp.bfloat16)
    gg = jnp.dot(xb, fcg_ref[...].astype(jnp.bfloat16),
                 preferred_element_type=jnp.float32)
    hh = jnp.dot(xb, fcx_ref[...].astype(jnp.bfloat16),
                 preferred_element_type=jnp.float32)
    act = ((gg * jax.nn.sigmoid(gg)) * hh).astype(jnp.bfloat16)
    contrib = jnp.dot(act, proj_ref[...].astype(jnp.bfloat16),
                      preferred_element_type=jnp.float32)

    @pl.when(h == 0)
    def _():
        out_ref[...] = contrib

    @pl.when(h != 0)
    def _():
        out_ref[...] += contrib


def kernel(x, shared_fc, shared_proj, experts_fc, experts_proj, gate_w,
           expert_bias):
    Bq, Tq, C = x.shape
    E = experts_fc.shape[0]
    HID = experts_proj.shape[1]
    K = _TOP_K
    N = Bq * Tq
    S = N * K
    i32 = jnp.int32

    TM = min(512, S)
    HB = min(1024, HID)
    assert S % TM == 0 and HID % HB == 0
    NT = S // TM
    NH = HID // HB
    G = NT + E - 1

    flat_x = x.reshape(N, C)

    logits = flat_x @ gate_w + expert_bias
    gw = jax.nn.sigmoid(logits)
    top_w, top_i = jax.lax.top_k(gw, K)
    top_w = top_w / jnp.sum(top_w, axis=-1, keepdims=True)
    top_i = top_i.astype(i32)

    onehot = jax.nn.one_hot(top_i, E, dtype=jnp.float32).sum(axis=1)
    csum = jnp.cumsum(onehot, axis=0)
    rank = jnp.take_along_axis((csum - onehot), top_i, axis=1)
    sizes = csum[-1].astype(i32)
    offsets = jnp.concatenate(
        [jnp.zeros((1,), i32), jnp.cumsum(sizes).astype(i32)])
    pos = jnp.take(offsets, top_i) + rank.astype(i32)

    pos_flat = pos.reshape(-1)
    tok_sorted = jnp.zeros((S,), i32).at[pos_flat].set(
        jnp.repeat(jnp.arange(N, dtype=i32), K))
    x_sorted = jnp.take(flat_x, tok_sorted, axis=0)

    first_tile = offsets[:E] // TM
    last_tile = (offsets[1:] - 1) // TM
    n_t = jnp.where(sizes > 0, last_tile - first_tile + 1, 0).astype(i32)
    cum_nt = jnp.cumsum(n_t)
    items_before = cum_nt - n_t
    total = cum_nt[-1]

    i = jnp.arange(G, dtype=i32)
    e_of = jnp.searchsorted(cum_nt, i, side='right').astype(i32)
    valid = i < total
    e_idx = jnp.minimum(e_of, E - 1)
    tile_ids = jnp.where(valid, first_tile[e_idx] + (i - items_before[e_idx]),
                         NT - 1).astype(i32)
    expert_ids = jnp.where(valid, e_idx, 0).astype(i32)
    off_arr = jnp.where(valid, offsets[e_idx], S).astype(i32)
    end_arr = jnp.where(valid, offsets[e_idx + 1], S).astype(i32)

    out_sorted = pl.pallas_call(
        functools.partial(_gmm_kernel, tm=TM),
        grid_spec=pltpu.PrefetchScalarGridSpec(
            num_scalar_prefetch=4,
            grid=(G, NH),
            in_specs=[
                pl.BlockSpec((TM, C),
                             lambda g, h, off, end, tid, eid: (tid[g], 0)),
                pl.BlockSpec((1, C, HB),
                             lambda g, h, off, end, tid, eid: (eid[g], 0, h)),
                pl.BlockSpec((1, C, HB),
                             lambda g, h, off, end, tid, eid:
                             (eid[g], 0, h + NH)),
                pl.BlockSpec((1, HB, C),
                             lambda g, h, off, end, tid, eid: (eid[g], h, 0)),
            ],
            out_specs=pl.BlockSpec((TM, C),
                                   lambda g, h, off, end, tid, eid:
                                   (tid[g], 0)),
        ),
        out_shape=jax.ShapeDtypeStruct((S, C), jnp.float32),
    )(off_arr, end_arr, tile_ids, expert_ids, x_sorted,
      experts_fc, experts_fc, experts_proj)

    TMS = min(1024, N)
    NTS = N // TMS
    HBS = min(1024, HID)
    NHS = HID // HBS
    shared_out = pl.pallas_call(
        _dense_ffn_kernel,
        grid=(NTS, NHS),
        in_specs=[
            pl.BlockSpec((TMS, C), lambda t, h: (t, 0)),
            pl.BlockSpec((C, HBS), lambda t, h: (0, h)),
            pl.BlockSpec((C, HBS), lambda t, h: (0, h + NHS)),
            pl.BlockSpec((HBS, C), lambda t, h: (h, 0)),
        ],
        out_specs=pl.BlockSpec((TMS, C), lambda t, h: (t, 0)),
        out_shape=jax.ShapeDtypeStruct((N, C), jnp.float32),
    )(flat_x, shared_fc, shared_fc, shared_proj)

    routed = jnp.sum(out_sorted[pos] * top_w[..., None], axis=1)

    return (shared_out + routed).reshape(Bq, Tq, C)


# speedup vs baseline: 1.3203x; 1.0862x over previous
import functools

import jax
import jax.numpy as jnp
from jax.experimental import pallas as pl
from jax.experimental.pallas import tpu as pltpu

_TOP_K = 4


def _gmm_kernel(off_r, end_r, tid_r, eid_r, x_ref, fcg_ref, fcx_ref,
                proj_ref, out_ref, *, tm):
    g = pl.program_id(0)
    h = pl.program_id(1)
    xb = x_ref[...].astype(jnp.bfloat16)
    gg = jnp.dot(xb, fcg_ref[0].astype(jnp.bfloat16),
                 preferred_element_type=jnp.float32)
    hh = jnp.dot(xb, fcx_ref[0].astype(jnp.bfloat16),
                 preferred_element_type=jnp.float32)
    act = (gg * jax.nn.sigmoid(gg)) * hh
    row = tid_r[g] * tm + jax.lax.broadcasted_iota(jnp.int32, (tm, 1), 0)
    mask = (row >= off_r[g]) & (row < end_r[g])
    act = jnp.where(mask, act, 0.0).astype(jnp.bfloat16)
    contrib = jnp.dot(act, proj_ref[0].astype(jnp.bfloat16),
                      preferred_element_type=jnp.float32)
    prev_tid = tid_r[jnp.maximum(g - 1, 0)]
    first = (h == 0) & ((g == 0) | (tid_r[g] != prev_tid))

    @pl.when(first)
    def _():
        out_ref[...] = contrib

    @pl.when(jnp.logical_not(first))
    def _():
        out_ref[...] += contrib


def _dense_ffn_kernel(x_ref, fcg_ref, fcx_ref, proj_ref, out_ref):
    h = pl.program_id(1)
    xb = x_ref[...].astype(jnp.bfloat16)
    gg = jnp.dot(xb, fcg_ref[...].astype(jnp.bfloat16),
                 preferred_element_type=jnp.float32)
    hh = jnp.dot(xb, fcx_ref[...].astype(jnp.bfloat16),
                 preferred_element_type=jnp.float32)
    act = ((gg * jax.nn.sigmoid(gg)) * hh).astype(jnp.bfloat16)
    contrib = jnp.dot(act, proj_ref[...].astype(jnp.bfloat16),
                      preferred_element_type=jnp.float32)

    @pl.when(h == 0)
    def _():
        out_ref[...] = contrib

    @pl.when(h != 0)
    def _():
        out_ref[...] += contrib


def kernel(x, shared_fc, shared_proj, experts_fc, experts_proj, gate_w,
           expert_bias):
    Bq, Tq, C = x.shape
    E = experts_fc.shape[0]
    HID = experts_proj.shape[1]
    K = _TOP_K
    N = Bq * Tq
    S = N * K
    i32 = jnp.int32

    TM = min(512, S)
    HB = min(1024, HID)
    assert S % TM == 0 and HID % HB == 0
    NT = S // TM
    NH = HID // HB
    G = NT + E - 1

    flat_x = x.reshape(N, C)

    logits = flat_x @ gate_w + expert_bias
    gw = jax.nn.sigmoid(logits)
    top_w, top_i = jax.lax.top_k(gw, K)
    top_w = top_w / jnp.sum(top_w, axis=-1, keepdims=True)
    top_i = top_i.astype(i32)

    onehot = jax.nn.one_hot(top_i, E, dtype=jnp.float32).sum(axis=1)
    csum = jnp.cumsum(onehot, axis=0)
    rank = jnp.take_along_axis((csum - onehot), top_i, axis=1)
    sizes = csum[-1].astype(i32)
    offsets = jnp.concatenate(
        [jnp.zeros((1,), i32), jnp.cumsum(sizes).astype(i32)])
    pos = jnp.take(offsets, top_i) + rank.astype(i32)

    pos_flat = pos.reshape(-1)
    tok_sorted = jnp.zeros((S,), i32).at[pos_flat].set(
        jnp.repeat(jnp.arange(N, dtype=i32), K))
    x_sorted = jnp.take(flat_x, tok_sorted, axis=0)

    first_tile = offsets[:E] // TM
    last_tile = (offsets[1:] - 1) // TM
    n_t = jnp.where(sizes > 0, last_tile - first_tile + 1, 0).astype(i32)
    cum_nt = jnp.cumsum(n_t)
    items_before = cum_nt - n_t
    total = cum_nt[-1]

    i = jnp.arange(G, dtype=i32)
    e_of = jnp.searchsorted(cum_nt, i, side='right').astype(i32)
    valid = i < total
    e_idx = jnp.minimum(e_of, E - 1)
    tile_ids = jnp.where(valid, first_tile[e_idx] + (i - items_before[e_idx]),
                         NT - 1).astype(i32)
    expert_ids = jnp.where(valid, e_idx, 0).astype(i32)
    off_arr = jnp.where(valid, offsets[e_idx], S).astype(i32)
    end_arr = jnp.where(valid, offsets[e_idx + 1], S).astype(i32)

    out_sorted = pl.pallas_call(
        functools.partial(_gmm_kernel, tm=TM),
        grid_spec=pltpu.PrefetchScalarGridSpec(
            num_scalar_prefetch=4,
            grid=(G, NH),
            in_specs=[
                pl.BlockSpec((TM, C),
                             lambda g, h, off, end, tid, eid: (tid[g], 0)),
                pl.BlockSpec((1, C, HB),
                             lambda g, h, off, end, tid, eid: (eid[g], 0, h)),
                pl.BlockSpec((1, C, HB),
                             lambda g, h, off, end, tid, eid:
                             (eid[g], 0, h + NH)),
                pl.BlockSpec((1, HB, C),
                             lambda g, h, off, end, tid, eid: (eid[g], h, 0)),
            ],
            out_specs=pl.BlockSpec((TM, C),
                                   lambda g, h, off, end, tid, eid:
                                   (tid[g], 0)),
        ),
        out_shape=jax.ShapeDtypeStruct((S, C), jnp.float32),
    )(off_arr, end_arr, tile_ids, expert_ids, x_sorted,
      experts_fc, experts_fc, experts_proj)

    TMS = min(1024, N)
    NTS = N // TMS
    HBS = min(1024, HID)
    NHS = HID // HBS
    shared_out = pl.pallas_call(
        _dense_ffn_kernel,
        grid=(NTS, NHS),
        in_specs=[
            pl.BlockSpec((TMS, C), lambda t, h: (t, 0)),
            pl.BlockSpec((C, HBS), lambda t, h: (0, h)),
            pl.BlockSpec((C, HBS), lambda t, h: (0, h + NHS)),
            pl.BlockSpec((HBS, C), lambda t, h: (h, 0)),
        ],
        out_specs=pl.BlockSpec((TMS, C), lambda t, h: (t, 0)),
        out_shape=jax.ShapeDtypeStruct((N, C), jnp.float32),
    )(flat_x, shared_fc, shared_fc, shared_proj)

    acc = shared_out
    for k in range(K):
        acc = acc + out_sorted[pos[:, k]] * top_w[:, k:k + 1]

    return acc.reshape(Bq, Tq, C)


# speedup vs baseline: 1.4207x; 1.0760x over previous
import functools

import jax
import jax.numpy as jnp
from jax import lax
from jax.experimental import pallas as pl
from jax.experimental.pallas import tpu as pltpu
from jax.experimental.pallas import tpu_sc as plsc

_TOP_K = 4


def _make_sc_row_gather(n_rows, n_cols, dtype):
    info = plsc.get_sparse_core_info()
    nw = info.num_cores * info.num_subcores
    b_per_w = n_rows // nw
    chunk = min(64, b_per_w)
    n_chunks = b_per_w // chunk
    mesh = plsc.VectorSubcoreMesh(core_axis_name="c", subcore_axis_name="s")

    @functools.partial(
        pl.kernel, mesh=mesh,
        out_type=jax.ShapeDtypeStruct((n_rows, n_cols), dtype),
        scratch_types=[
            pltpu.VMEM((chunk,), jnp.int32),
            pltpu.VMEM((chunk, n_cols), dtype),
            pltpu.SemaphoreType.DMA,
        ],
    )
    def gather_k(table_hbm, idx_hbm, out_hbm, idx_v, rows_v, sem):
        wid = lax.axis_index("s") * info.num_cores + lax.axis_index("c")
        base = wid * b_per_w
        for j in range(n_chunks):
            off = base + j * chunk
            pltpu.sync_copy(idx_hbm.at[pl.ds(off, chunk)], idx_v)
            pltpu.async_copy(table_hbm.at[idx_v], rows_v, sem).wait()
            pltpu.sync_copy(rows_v, out_hbm.at[pl.ds(off, chunk)])

    return gather_k


def _gmm_kernel(off_r, end_r, tid_r, eid_r, x_ref, fcg_ref, fcx_ref,
                proj_ref, out_ref, *, tm):
    g = pl.program_id(0)
    h = pl.program_id(1)
    xb = x_ref[...].astype(jnp.bfloat16)
    gg = jnp.dot(xb, fcg_ref[0].astype(jnp.bfloat16),
                 preferred_element_type=jnp.float32)
    hh = jnp.dot(xb, fcx_ref[0].astype(jnp.bfloat16),
                 preferred_element_type=jnp.float32)
    act = (gg * jax.nn.sigmoid(gg)) * hh
    row = tid_r[g] * tm + jax.lax.broadcasted_iota(jnp.int32, (tm, 1), 0)
    mask = (row >= off_r[g]) & (row < end_r[g])
    act = jnp.where(mask, act, 0.0).astype(jnp.bfloat16)
    contrib = jnp.dot(act, proj_ref[0].astype(jnp.bfloat16),
                      preferred_element_type=jnp.float32)
    prev_tid = tid_r[jnp.maximum(g - 1, 0)]
    first = (h == 0) & ((g == 0) | (tid_r[g] != prev_tid))

    @pl.when(first)
    def _():
        out_ref[...] = contrib

    @pl.when(jnp.logical_not(first))
    def _():
        out_ref[...] += contrib


def _dense_ffn_kernel(x_ref, fcg_ref, fcx_ref, proj_ref, out_ref):
    h = pl.program_id(1)
    xb = x_ref[...].astype(jnp.bfloat16)
    gg = jnp.dot(xb, fcg_ref[...].astype(jnp.bfloat16),
                 preferred_element_type=jnp.float32)
    hh = jnp.dot(xb, fcx_ref[...].astype(jnp.bfloat16),
                 preferred_element_type=jnp.float32)
    act = ((gg * jax.nn.sigmoid(gg)) * hh).astype(jnp.bfloat16)
    contrib = jnp.dot(act, proj_ref[...].astype(jnp.bfloat16),
                      preferred_element_type=jnp.float32)

    @pl.when(h == 0)
    def _():
        out_ref[...] = contrib

    @pl.when(h != 0)
    def _():
        out_ref[...] += contrib


def kernel(x, shared_fc, shared_proj, experts_fc, experts_proj, gate_w,
           expert_bias):
    Bq, Tq, C = x.shape
    E = experts_fc.shape[0]
    HID = experts_proj.shape[1]
    K = _TOP_K
    N = Bq * Tq
    S = N * K
    i32 = jnp.int32

    TM = min(512, S)
    HB = min(1024, HID)
    assert S % TM == 0 and HID % HB == 0
    NT = S // TM
    NH = HID // HB
    G = NT + E - 1

    flat_x = x.reshape(N, C)

    logits = flat_x @ gate_w + expert_bias
    gw = jax.nn.sigmoid(logits)
    top_w, top_i = jax.lax.top_k(gw, K)
    top_w = top_w / jnp.sum(top_w, axis=-1, keepdims=True)
    top_i = top_i.astype(i32)

    onehot = jax.nn.one_hot(top_i, E, dtype=jnp.float32).sum(axis=1)
    csum = jnp.cumsum(onehot, axis=0)
    rank = jnp.take_along_axis((csum - onehot), top_i, axis=1)
    sizes = csum[-1].astype(i32)
    offsets = jnp.concatenate(
        [jnp.zeros((1,), i32), jnp.cumsum(sizes).astype(i32)])
    pos = jnp.take(offsets, top_i) + rank.astype(i32)

    pos_flat = pos.reshape(-1)
    tok_sorted = jnp.zeros((S,), i32).at[pos_flat].set(
        jnp.repeat(jnp.arange(N, dtype=i32), K))
    x_sorted = _make_sc_row_gather(S, C, flat_x.dtype)(flat_x, tok_sorted)

    first_tile = offsets[:E] // TM
    last_tile = (offsets[1:] - 1) // TM
    n_t = jnp.where(sizes > 0, last_tile - first_tile + 1, 0).astype(i32)
    cum_nt = jnp.cumsum(n_t)
    items_before = cum_nt - n_t
    total = cum_nt[-1]

    i = jnp.arange(G, dtype=i32)
    e_of = jnp.searchsorted(cum_nt, i, side='right').astype(i32)
    valid = i < total
    e_idx = jnp.minimum(e_of, E - 1)
    tile_ids = jnp.where(valid, first_tile[e_idx] + (i - items_before[e_idx]),
                         NT - 1).astype(i32)
    expert_ids = jnp.where(valid, e_idx, 0).astype(i32)
    off_arr = jnp.where(valid, offsets[e_idx], S).astype(i32)
    end_arr = jnp.where(valid, offsets[e_idx + 1], S).astype(i32)

    out_sorted = pl.pallas_call(
        functools.partial(_gmm_kernel, tm=TM),
        grid_spec=pltpu.PrefetchScalarGridSpec(
            num_scalar_prefetch=4,
            grid=(G, NH),
            in_specs=[
                pl.BlockSpec((TM, C),
                             lambda g, h, off, end, tid, eid: (tid[g], 0)),
                pl.BlockSpec((1, C, HB),
                             lambda g, h, off, end, tid, eid: (eid[g], 0, h)),
                pl.BlockSpec((1, C, HB),
                             lambda g, h, off, end, tid, eid:
                             (eid[g], 0, h + NH)),
                pl.BlockSpec((1, HB, C),
                             lambda g, h, off, end, tid, eid: (eid[g], h, 0)),
            ],
            out_specs=pl.BlockSpec((TM, C),
                                   lambda g, h, off, end, tid, eid:
                                   (tid[g], 0)),
        ),
        out_shape=jax.ShapeDtypeStruct((S, C), jnp.float32),
    )(off_arr, end_arr, tile_ids, expert_ids, x_sorted,
      experts_fc, experts_fc, experts_proj)

    TMS = min(1024, N)
    NTS = N // TMS
    HBS = min(1024, HID)
    NHS = HID // HBS
    shared_out = pl.pallas_call(
        _dense_ffn_kernel,
        grid=(NTS, NHS),
        in_specs=[
            pl.BlockSpec((TMS, C), lambda t, h: (t, 0)),
            pl.BlockSpec((C, HBS), lambda t, h: (0, h)),
            pl.BlockSpec((C, HBS), lambda t, h: (0, h + NHS)),
            pl.BlockSpec((HBS, C), lambda t, h: (h, 0)),
        ],
        out_specs=pl.BlockSpec((TMS, C), lambda t, h: (t, 0)),
        out_shape=jax.ShapeDtypeStruct((N, C), jnp.float32),
    )(flat_x, shared_fc, shared_fc, shared_proj)

    acc = shared_out
    for k in range(K):
        acc = acc + out_sorted[pos[:, k]] * top_w[:, k:k + 1]

    return acc.reshape(Bq, Tq, C)
